# Initial kernel scaffold; baseline (speedup 1.0000x reference)
import jax
import jax.numpy as jnp
from jax.experimental import pallas as pl


def kernel(token_ids, emb_table, pos_table, W, b, gamma, beta):
    raise NotImplementedError("write your pallas kernel here")


# speedup vs baseline: 12.2505x; 12.2505x over previous
import functools

import jax
import jax.numpy as jnp
from jax import lax
from jax.experimental import pallas as pl
from jax.experimental.pallas import tpu as pltpu
from jax.experimental.pallas import tpu_sc as plsc

_NC, _NS, _LANES = 2, 16, 16
_NW = _NC * _NS


def _sc_pool(tok2d, emb_table, B, Lseq):
    TR_TOTAL, TW = tok2d.shape
    D = emb_table.shape[1]
    RPT = TW // Lseq
    TR = TR_TOTAL // _NW
    KSUB = 4
    NCHUNK = TR // KSUB
    BPW = B // _NW
    ROWS_PER_CHUNK = KSUB * RPT
    NQ = D // _LANES

    mesh = plsc.VectorSubcoreMesh(core_axis_name="c", subcore_axis_name="s")

    @functools.partial(
        pl.kernel,
        out_type=jax.ShapeDtypeStruct((B, D), jnp.float32),
        mesh=mesh,
        compiler_params=pltpu.CompilerParams(use_tc_tiling_on_sc=False),
        scratch_types=[
            pltpu.VMEM((KSUB, TW), jnp.int32),
            pltpu.VMEM((KSUB, TW, D), jnp.float32),
            pltpu.VMEM((BPW, D), jnp.float32),
            pltpu.SemaphoreType.DMA,
        ],
    )
    def pool_kernel(tok_hbm, tab_hbm, out_hbm, idx_v, rows_v, acc_v, gsem):
        wid = lax.axis_index("s") * _NC + lax.axis_index("c")
        row_base = wid * BPW
        trow_base = wid * TR

        @pl.loop(0, NCHUNK)
        def _chunk(ci):
            tr0 = trow_base + ci * KSUB
            pltpu.sync_copy(tok_hbm.at[pl.ds(tr0, KSUB)], idx_v)
            cps = [
                pltpu.async_copy(tab_hbm.at[idx_v.at[k]], rows_v.at[k], gsem)
                for k in range(KSUB)
            ]
            for cp in cps:
                cp.wait()
            for k in range(KSUB):
                for h in range(RPT):
                    def jbody(j, carry, _k=k, _h=h):
                        return tuple(
                            carry[q]
                            + rows_v[_k, _h * Lseq + j, pl.ds(q * _LANES, _LANES)]
                            for q in range(NQ)
                        )
                    zero = jnp.zeros((_LANES,), jnp.float32)
                    accs = lax.fori_loop(0, Lseq, jbody, (zero,) * NQ)
                    lr = ci * ROWS_PER_CHUNK + RPT * k + h
                    for q in range(NQ):
                        acc_v[lr, pl.ds(q * _LANES, _LANES)] = accs[q]

        pltpu.sync_copy(acc_v, out_hbm.at[pl.ds(row_base, BPW)])

    return pool_kernel(tok2d, emb_table)


def _tc_head(sums, pos_table, W, b, gamma, beta, Lseq):
    B, D = sums.shape
    ML = pos_table.shape[0]
    O = W.shape[1]
    BB = 2048
    inv_l = 1.0 / Lseq

    def body(s_ref, pos_ref, w_ref, b_ref, g_ref, be_ref, o_ref):
        x = s_ref[...] * inv_l
        pos = pos_ref[...]
        ridx = lax.broadcasted_iota(jnp.int32, pos.shape, 0)
        pos_mean = jnp.sum(jnp.where(ridx < Lseq, pos, 0.0), axis=0,
                           keepdims=True) * inv_l
        y = (jnp.dot(x + pos_mean, w_ref[...],
                     preferred_element_type=jnp.float32) + b_ref[...])
        mu = jnp.mean(y, axis=1, keepdims=True)
        yc = y - mu
        var = jnp.mean(yc * yc, axis=1, keepdims=True)
        o_ref[...] = g_ref[...] * yc * lax.rsqrt(var + 1e-5) + be_ref[...]

    return pl.pallas_call(
        body,
        grid=(B // BB,),
        in_specs=[
            pl.BlockSpec((BB, D), lambda i: (i, 0)),
            pl.BlockSpec((ML, D), lambda i: (0, 0)),
            pl.BlockSpec((D, O), lambda i: (0, 0)),
            pl.BlockSpec((1, O), lambda i: (0, 0)),
            pl.BlockSpec((1, O), lambda i: (0, 0)),
            pl.BlockSpec((1, O), lambda i: (0, 0)),
        ],
        out_specs=pl.BlockSpec((BB, O), lambda i: (i, 0)),
        out_shape=jax.ShapeDtypeStruct((B, O), jnp.float32),
    )(sums, pos_table, W, b.reshape(1, O), gamma.reshape(1, O),
      beta.reshape(1, O))


def kernel(token_ids, emb_table, pos_table, W, b, gamma, beta):
    B, Lseq = token_ids.shape
    TW = 2 * Lseq
    assert (B * Lseq) % TW == 0 and (B * Lseq // TW) % (_NW * 4) == 0
    tok2d = token_ids.reshape(-1, TW)
    sums = _sc_pool(tok2d, emb_table, B, Lseq)
    return _tc_head(sums, pos_table, W, b, gamma, beta, Lseq)


# speedup vs baseline: 19.6141x; 1.6011x over previous
import functools

import jax
import jax.numpy as jnp
from jax import lax
from jax.experimental import pallas as pl
from jax.experimental.pallas import tpu as pltpu
from jax.experimental.pallas import tpu_sc as plsc

_NC, _NS, _LANES = 2, 16, 16
_NW = _NC * _NS


def _sc_pool(tok2d, emb_table, B, Lseq):
    TR_TOTAL, TW = tok2d.shape
    D = emb_table.shape[1]
    RPT = TW // Lseq
    TR = TR_TOTAL // _NW
    KSUB = 4
    NCHUNK = TR // KSUB
    BPW = B // _NW
    ROWS_PER_CHUNK = KSUB * RPT
    NQ = D // _LANES

    mesh = plsc.VectorSubcoreMesh(core_axis_name="c", subcore_axis_name="s")

    @functools.partial(
        pl.kernel,
        out_type=jax.ShapeDtypeStruct((B, D), jnp.float32),
        mesh=mesh,
        compiler_params=pltpu.CompilerParams(use_tc_tiling_on_sc=False),
        scratch_types=[
            pltpu.VMEM((TR, TW), jnp.int32),
            pltpu.VMEM((2, KSUB, TW, D), jnp.float32),
            pltpu.VMEM((BPW, D), jnp.float32),
            pltpu.SemaphoreType.DMA,
            pltpu.SemaphoreType.DMA,
        ],
    )
    def pool_kernel(tok_hbm, tab_hbm, out_hbm, idx_v, rows_v, acc_v, sem0,
                    sem1):
        wid = lax.axis_index("s") * _NC + lax.axis_index("c")
        row_base = wid * BPW
        trow_base = wid * TR
        sems = (sem0, sem1)

        pltpu.sync_copy(tok_hbm.at[pl.ds(trow_base, TR)], idx_v)

        def fire(ci, slot):
            for k in range(KSUB):
                pltpu.async_copy(
                    tab_hbm.at[idx_v.at[ci * KSUB + k]],
                    rows_v.at[slot, k], sems[slot])

        def drain(slot):
            for k in range(KSUB):
                pltpu.make_async_copy(
                    tab_hbm.at[idx_v.at[k]],
                    rows_v.at[slot, k], sems[slot]).wait()

        def accumulate(ci, slot):
            @pl.loop(0, ROWS_PER_CHUNK)
            def _row(rh):
                k = rh // RPT
                h = rh % RPT
                base = h * Lseq
                accs = [rows_v[slot, k, base, pl.ds(q * _LANES, _LANES)]
                        for q in range(NQ)]
                for j in range(1, Lseq):
                    for q in range(NQ):
                        accs[q] = accs[q] + rows_v[
                            slot, k, base + j, pl.ds(q * _LANES, _LANES)]
                lr = ci * ROWS_PER_CHUNK + rh
                for q in range(NQ):
                    acc_v[lr, pl.ds(q * _LANES, _LANES)] = accs[q]

        fire(0, 0)

        @pl.loop(0, NCHUNK, step=2)
        def _chunk(ci):
            fire(ci + 1, 1)
            drain(0)
            accumulate(ci, 0)

            @pl.when(ci + 2 < NCHUNK)
            def _():
                fire(ci + 2, 0)

            drain(1)
            accumulate(ci + 1, 1)

        pltpu.sync_copy(acc_v, out_hbm.at[pl.ds(row_base, BPW)])

    return pool_kernel(tok2d, emb_table)


def _tc_head(sums, pos_table, W, b, gamma, beta, Lseq):
    B, D = sums.shape
    ML = pos_table.shape[0]
    O = W.shape[1]
    BB = 2048
    inv_l = 1.0 / Lseq

    def body(s_ref, pos_ref, w_ref, b_ref, g_ref, be_ref, o_ref):
        x = s_ref[...] * inv_l
        pos = pos_ref[...]
        ridx = lax.broadcasted_iota(jnp.int32, pos.shape, 0)
        pos_mean = jnp.sum(jnp.where(ridx < Lseq, pos, 0.0), axis=0,
                           keepdims=True) * inv_l
        y = (jnp.dot(x + pos_mean, w_ref[...],
                     preferred_element_type=jnp.float32) + b_ref[...])
        mu = jnp.mean(y, axis=1, keepdims=True)
        yc = y - mu
        var = jnp.mean(yc * yc, axis=1, keepdims=True)
        o_ref[...] = g_ref[...] * yc * lax.rsqrt(var + 1e-5) + be_ref[...]

    return pl.pallas_call(
        body,
        grid=(B // BB,),
        in_specs=[
            pl.BlockSpec((BB, D), lambda i: (i, 0)),
            pl.BlockSpec((ML, D), lambda i: (0, 0)),
            pl.BlockSpec((D, O), lambda i: (0, 0)),
            pl.BlockSpec((1, O), lambda i: (0, 0)),
            pl.BlockSpec((1, O), lambda i: (0, 0)),
            pl.BlockSpec((1, O), lambda i: (0, 0)),
        ],
        out_specs=pl.BlockSpec((BB, O), lambda i: (i, 0)),
        out_shape=jax.ShapeDtypeStruct((B, O), jnp.float32),
    )(sums, pos_table, W, b.reshape(1, O), gamma.reshape(1, O),
      beta.reshape(1, O))


def kernel(token_ids, emb_table, pos_table, W, b, gamma, beta):
    B, Lseq = token_ids.shape
    TW = 2 * Lseq
    assert (B * Lseq) % TW == 0 and (B * Lseq // TW) % (_NW * 4) == 0
    tok2d = token_ids.reshape(-1, TW)
    sums = _sc_pool(tok2d, emb_table, B, Lseq)
    return _tc_head(sums, pos_table, W, b, gamma, beta, Lseq)
